# baseline (speedup 1.0000x reference)
import functools

import jax
import jax.numpy as jnp
from jax import lax
from jax.experimental import pallas as pl
from jax.experimental.pallas import tpu as pltpu
from jax.experimental.pallas import tpu_sc as plsc

_D = 64
_H = 200
_B = 4096
_OUT = 128
_NC = 2
_NS = 16
_NW = _NC * _NS
_BPW = _B // _NW
_S0, _S1 = 104, 96
_NBUF = 4
_UNROLL = 8


def _pool_body(idx_hbm, table_hbm, out_hbm, idx_v, rows_v, out_v, sem):
    wid = lax.axis_index("s") * _NC + lax.axis_index("c")
    base = wid * _BPW
    pltpu.sync_copy(idx_hbm.at[pl.ds(base, _BPW)], idx_v)

    def issue(r, slot):
        pltpu.async_copy(
            table_hbm.at[idx_v.at[r, pl.ds(0, _S0)]],
            rows_v.at[slot, pl.ds(0, _S0)],
            sem,
        )
        pltpu.async_copy(
            table_hbm.at[idx_v.at[r, pl.ds(_S0, _S1)]],
            rows_v.at[slot, pl.ds(_S0, _S1)],
            sem,
        )

    def wait(slot):
        pltpu.make_async_copy(
            table_hbm.at[idx_v.at[0, pl.ds(0, _S0)]],
            rows_v.at[slot, pl.ds(0, _S0)],
            sem,
        ).wait()
        pltpu.make_async_copy(
            table_hbm.at[idx_v.at[0, pl.ds(_S0, _S1)]],
            rows_v.at[slot, pl.ds(_S0, _S1)],
            sem,
        ).wait()

    def reduce_into(slot, r):
        zero = jnp.zeros((16,), jnp.float32)

        def body(g, accs):
            j = g * _UNROLL
            new = []
            for k in range(_D // 16):
                c = pl.ds(k * 16, 16)
                x = [rows_v[slot, j + u, c] for u in range(_UNROLL)]
                t01 = x[0] + x[1]
                t23 = x[2] + x[3]
                t45 = x[4] + x[5]
                t67 = x[6] + x[7]
                new.append(accs[k] + ((t01 + t23) + (t45 + t67)))
            return tuple(new)

        accs = lax.fori_loop(0, _H // _UNROLL, body, (zero,) * (_D // 16))
        for k in range(_D // 16):
            out_v[r, pl.ds(k * 16, 16)] = accs[k]

    for p in range(_NBUF):
        issue(p, p)

    def outer(g, carry):
        for slot in range(_NBUF):
            r = g * _NBUF + slot
            wait(slot)
            reduce_into(slot, r)
            nxt = r + _NBUF

            @pl.when(nxt < _BPW)
            def _():
                issue(nxt, slot)

        return carry

    lax.fori_loop(0, _BPW // _NBUF, outer, 0)
    pltpu.sync_copy(out_v, out_hbm.at[pl.ds(base, _BPW)])


def _make_pool():
    mesh = plsc.VectorSubcoreMesh(core_axis_name="c", subcore_axis_name="s")
    return functools.partial(
        pl.kernel,
        mesh=mesh,
        out_type=jax.ShapeDtypeStruct((_B, _D), jnp.float32),
        scratch_types=[
            pltpu.VMEM((_BPW, _H), jnp.int32),
            pltpu.VMEM((_NBUF, _H, _D), jnp.float32),
            pltpu.VMEM((_BPW, _D), jnp.float32),
            pltpu.SemaphoreType.DMA,
        ],
        compiler_params=pltpu.CompilerParams(use_tc_tiling_on_sc=False),
    )(_pool_body)


_POOL = _make_pool()


def _linear_body(x_ref, w_ref, b_ref, o_ref):
    x = x_ref[...] * (1.0 / _H)
    o_ref[...] = (
        jnp.dot(x, w_ref[...], preferred_element_type=jnp.float32) + b_ref[...]
    )


_BLK = 1024


def _linear(x, w, b):
    return pl.pallas_call(
        _linear_body,
        grid=(_B // _BLK,),
        in_specs=[
            pl.BlockSpec((_BLK, _D), lambda i: (i, 0)),
            pl.BlockSpec((_D, _OUT), lambda i: (0, 0)),
            pl.BlockSpec((1, _OUT), lambda i: (0, 0)),
        ],
        out_specs=pl.BlockSpec((_BLK, _OUT), lambda i: (i, 0)),
        out_shape=jax.ShapeDtypeStruct((_B, _OUT), jnp.float32),
    )(x, w, b.reshape(1, _OUT))


def kernel(word_indices, embedding, W, b):
    pooled = _POOL(word_indices.astype(jnp.int32), embedding)
    return _linear(pooled, W, b)
